# baseline (speedup 1.0000x reference)
import jax
import jax.numpy as jnp
from jax import lax
from jax.experimental import pallas as pl
from jax.experimental.pallas import tpu as pltpu
from jax.experimental.pallas import tpu_sc as plsc

N = 10000
D = 128
H = 64
HP = 128
G = 128
E = 320000

NC = 2
NS = 16
NW = NC * NS
STREAM = 128
KSTREAMS = 80
NBUF = 3
NRING = 6
EPAD = NW * KSTREAMS * STREAM
NACC = 10016
RPS = 632
RPSL = NACC - (NS - 1) * RPS

RBLK = 2000
NBLK = N // RBLK
BN_EPS = 1e-5

_PREC = jax.lax.Precision.HIGHEST


def _sc_agg_body(y_hbm, srcdst_hbm, zeros_hbm, out_hbm,
                 ring, b0, b1, b2, acc_sh,
                 isems, gsems, ssems):
    cid = lax.axis_index("c")
    sid = lax.axis_index("s")
    wid = sid * NC + cid
    my_idx = srcdst_hbm.at[wid]

    bufs = [b0, b1, b2]

    @pl.when(sid < NS - 1)
    def _():
        pltpu.sync_copy(zeros_hbm, acc_sh.at[pl.ds(sid * RPS, RPS)])

    @pl.when(sid == NS - 1)
    def _():
        pltpu.sync_copy(zeros_hbm.at[pl.ds(0, RPSL)],
                        acc_sh.at[pl.ds((NS - 1) * RPS, RPSL)])

    plsc.subcore_barrier()

    def idxcopy(j, s):
        pltpu.async_copy(my_idx.at[j], ring.at[s], isems.at[s])

    def wait_idx(s):
        pltpu.make_async_copy(my_idx.at[0], ring.at[s], isems.at[s]).wait()

    def gather(s, t):
        pltpu.async_copy(y_hbm.at[ring.at[s, 0]], bufs[t], gsems.at[t])

    def wait_gather(t):
        pltpu.make_async_copy(y_hbm.at[ring.at[0, 0]], bufs[t],
                              gsems.at[t]).wait()

    def scat(s, t):
        pltpu.async_copy(bufs[t], acc_sh.at[ring.at[s, 1]], ssems.at[t],
                         add=True)

    def wait_scat(t):
        pltpu.make_async_copy(bufs[t], acc_sh.at[ring.at[0, 1]],
                              ssems.at[t]).wait()

    def step(j, k, sw=True, gat=True, idx=True):
        wait_gather(k % 3)
        if sw:
            wait_scat((k - 1) % 3)
        if gat:
            wait_idx((k + 2) % NRING)
            gather((k + 2) % NRING, (k + 2) % 3)
        scat(k % NRING, k % 3)
        if idx:
            idxcopy(j + 5, (k + 5) % NRING)

    for s in range(5):
        idxcopy(s, s)
    wait_idx(0)
    gather(0, 0)
    wait_idx(1)
    gather(1, 1)
    step(0, 0, sw=False)
    step(1, 1)

    @pl.loop(0, 12)
    def _(jj):
        j6 = 2 + jj * 6
        for k in range(6):
            step(j6 + k, 2 + k)

    for j in range(74, KSTREAMS):
        step(j, j, gat=(j + 2 < KSTREAMS), idx=(j + 5 < KSTREAMS))
    wait_scat((KSTREAMS - 1) % 3)

    plsc.subcore_barrier()

    @pl.when(sid < NS - 1)
    def _():
        pltpu.sync_copy(acc_sh.at[pl.ds(sid * RPS, RPS)],
                        out_hbm.at[cid].at[pl.ds(sid * RPS, RPS)])

    @pl.when(sid == NS - 1)
    def _():
        pltpu.sync_copy(acc_sh.at[pl.ds((NS - 1) * RPS, RPSL)],
                        out_hbm.at[cid].at[pl.ds((NS - 1) * RPS, RPSL)])


def _sc_agg(y, srcdst, zeros):
    mesh = plsc.VectorSubcoreMesh(core_axis_name="c", subcore_axis_name="s")
    kern = pl.kernel(
        _sc_agg_body,
        out_type=jax.ShapeDtypeStruct((NC, NACC, HP), jnp.float32),
        mesh=mesh,
        scratch_types=[
            pltpu.VMEM((NRING, 2, STREAM), jnp.int32),
            pltpu.VMEM((STREAM, HP), jnp.float32),
            pltpu.VMEM((STREAM, HP), jnp.float32),
            pltpu.VMEM((STREAM, HP), jnp.float32),
            pltpu.VMEM_SHARED((NACC, HP), jnp.float32),
            pltpu.SemaphoreType.DMA((NRING,)),
            pltpu.SemaphoreType.DMA((NBUF,)),
            pltpu.SemaphoreType.DMA((NBUF,)),
        ],
    )
    return kern(y, srcdst, zeros)


def _mlp_body(x_ref, agg_ref, w1_ref, b1_ref, w2_ref, b2_ref,
              r_ref, st_ref):
    i = pl.program_id(0)
    s = x_ref[...] + agg_ref[0] + agg_ref[1]
    h = jax.lax.dot_general(s, w1_ref[...], (((1,), (0,)), ((), ())),
                            precision=_PREC,
                            preferred_element_type=jnp.float32)
    h = jnp.maximum(h + b1_ref[...], 0.0)
    t = jax.lax.dot_general(h, w2_ref[...], (((1,), (0,)), ((), ())),
                            precision=_PREC,
                            preferred_element_type=jnp.float32)
    r = jnp.maximum(t + b2_ref[...], 0.0)
    col = jax.lax.broadcasted_iota(jnp.int32, (RBLK, H), 1)
    ones_lane = jnp.where(col == 0, 1.0, 0.0).astype(jnp.float32)
    r_ref[...] = jnp.concatenate([r, ones_lane], axis=1)

    @pl.when(i == 0)
    def _():
        st_ref[...] = jnp.zeros_like(st_ref)

    st_ref[0:1, :] += jnp.sum(r, axis=0, keepdims=True)
    st_ref[1:2, :] += jnp.sum(r * r, axis=0, keepdims=True)


def _mlp(x, agg, w1, b1, w2, b2):
    return pl.pallas_call(
        _mlp_body,
        grid=(NBLK,),
        in_specs=[
            pl.BlockSpec((RBLK, HP), lambda i: (i, 0)),
            pl.BlockSpec((NC, RBLK, HP), lambda i: (0, i, 0)),
            pl.BlockSpec((HP, H), lambda i: (0, 0)),
            pl.BlockSpec((1, H), lambda i: (0, 0)),
            pl.BlockSpec((H, H), lambda i: (0, 0)),
            pl.BlockSpec((1, H), lambda i: (0, 0)),
        ],
        out_specs=[
            pl.BlockSpec((RBLK, HP), lambda i: (i, 0)),
            pl.BlockSpec((8, H), lambda i: (0, 0)),
        ],
        out_shape=[
            jax.ShapeDtypeStruct((N, HP), jnp.float32),
            jax.ShapeDtypeStruct((8, H), jnp.float32),
        ],
    )(x, agg, w1, b1, w2, b2)


def _bn_coeffs(st_ref, gamma_ref, beta_ref):
    mu = st_ref[0:1, :] * (1.0 / N)
    var = st_ref[1:2, :] * (1.0 / N) - mu * mu
    c = gamma_ref[...] * jax.lax.rsqrt(var + BN_EPS)
    d = beta_ref[...] - mu * c
    return c, d


def _bn0_body(r_ref, st_ref, gamma_ref, beta_ref, w1_ref, z_ref, m_ref):
    c, d = _bn_coeffs(st_ref, gamma_ref, beta_ref)
    z_ref[...] = r_ref[:, :H] * c + d
    row = jax.lax.broadcasted_iota(jnp.int32, (H, H), 0)
    colj = jax.lax.broadcasted_iota(jnp.int32, (H, H), 1)
    eyec = jnp.where(row == colj, 1.0, 0.0).astype(jnp.float32) * c
    cw1 = jax.lax.dot_general(eyec, w1_ref[...], (((1,), (0,)), ((), ())),
                              precision=_PREC,
                              preferred_element_type=jnp.float32)
    dw1 = jax.lax.dot_general(d, w1_ref[...], (((1,), (0,)), ((), ())),
                              precision=_PREC,
                              preferred_element_type=jnp.float32)
    zpad = jnp.zeros((HP - H - 1, H), jnp.float32)
    m_ref[...] = jnp.concatenate([cw1, dw1, zpad], axis=0)


def _bn0(r, st, gamma, beta, w1):
    return pl.pallas_call(
        _bn0_body,
        grid=(NBLK,),
        in_specs=[
            pl.BlockSpec((RBLK, HP), lambda i: (i, 0)),
            pl.BlockSpec((8, H), lambda i: (0, 0)),
            pl.BlockSpec((1, H), lambda i: (0, 0)),
            pl.BlockSpec((1, H), lambda i: (0, 0)),
            pl.BlockSpec((H, H), lambda i: (0, 0)),
        ],
        out_specs=[
            pl.BlockSpec((RBLK, H), lambda i: (i, 0)),
            pl.BlockSpec((HP, H), lambda i: (0, 0)),
        ],
        out_shape=[
            jax.ShapeDtypeStruct((N, H), jnp.float32),
            jax.ShapeDtypeStruct((HP, H), jnp.float32),
        ],
    )(r, st, gamma, beta, w1)


def _final_body(r_ref, st_ref, gamma_ref, beta_ref, z0_ref, batch_ref,
                z_ref, g_ref):
    i = pl.program_id(0)
    c, d = _bn_coeffs(st_ref, gamma_ref, beta_ref)
    z1 = r_ref[:, :H] * c + d
    zc = jnp.concatenate([z0_ref[...], z1], axis=1)
    z_ref[...] = zc

    seg = batch_ref[...]
    cols = jax.lax.broadcasted_iota(jnp.int32, (RBLK, G), 1)
    p = jnp.where(seg == cols, 1.0, 0.0).astype(jnp.float32)
    gpart = jax.lax.dot_general(p, zc, (((0,), (0,)), ((), ())),
                                precision=_PREC,
                                preferred_element_type=jnp.float32)

    @pl.when(i == 0)
    def _():
        g_ref[...] = jnp.zeros_like(g_ref)

    g_ref[...] += gpart


def _final(r, st, gamma, beta, z0, batch2d):
    return pl.pallas_call(
        _final_body,
        grid=(NBLK,),
        in_specs=[
            pl.BlockSpec((RBLK, HP), lambda i: (i, 0)),
            pl.BlockSpec((8, H), lambda i: (0, 0)),
            pl.BlockSpec((1, H), lambda i: (0, 0)),
            pl.BlockSpec((1, H), lambda i: (0, 0)),
            pl.BlockSpec((RBLK, H), lambda i: (i, 0)),
            pl.BlockSpec((RBLK, 1), lambda i: (i, 0)),
        ],
        out_specs=[
            pl.BlockSpec((RBLK, 2 * H), lambda i: (i, 0)),
            pl.BlockSpec((G, 2 * H), lambda i: (0, 0)),
        ],
        out_shape=[
            jax.ShapeDtypeStruct((N, 2 * H), jnp.float32),
            jax.ShapeDtypeStruct((G, 2 * H), jnp.float32),
        ],
    )(r, st, gamma, beta, z0, batch2d)


def kernel(x, edge_index, batch, W1_0, b1_0, W2_0, b2_0, gamma_0, beta_0,
           W1_1, b1_1, W2_1, b2_1, gamma_1, beta_1):
    pad = EPAD - E
    pad_ar = jnp.arange(pad, dtype=jnp.int32)
    src_pad = (pad_ar * 37) % N
    dst_pad = N + (pad_ar % (NACC - N))
    srcm = jnp.concatenate([edge_index[0], src_pad]).reshape(
        NW, KSTREAMS, STREAM)
    dstm = jnp.concatenate([edge_index[1], dst_pad]).reshape(
        NW, KSTREAMS, STREAM)
    srcdst = jnp.stack([srcm, dstm], axis=2)
    zeros = jnp.zeros((RPS, HP), jnp.float32)
    batch2d = batch.reshape(N, 1)
    b1_0r, b2_0r = b1_0.reshape(1, H), b2_0.reshape(1, H)
    b1_1r, b2_1r = b1_1.reshape(1, H), b2_1.reshape(1, H)
    g0r, be0r = gamma_0.reshape(1, H), beta_0.reshape(1, H)
    g1r, be1r = gamma_1.reshape(1, H), beta_1.reshape(1, H)

    agg0 = _sc_agg(x, srcdst, zeros)
    r0p, st0 = _mlp(x, agg0, W1_0, b1_0r, W2_0, b2_0r)

    agg1 = _sc_agg(r0p, srcdst, zeros)
    z0, m0 = _bn0(r0p, st0, g0r, be0r, W1_1)
    r1p, st1 = _mlp(r0p, agg1, m0, b1_1r, W2_1, b2_1r)

    z_out, g_out = _final(r1p, st1, g1r, be1r, z0, batch2d)
    return (z_out, g_out)


# speedup vs baseline: 1.0374x; 1.0374x over previous
import jax
import jax.numpy as jnp
from jax import lax
from jax.experimental import pallas as pl
from jax.experimental.pallas import tpu as pltpu
from jax.experimental.pallas import tpu_sc as plsc

N = 10000
D = 128
H = 64
HP = 128
G = 128
E = 320000

NC = 2
NS = 16
NW = NC * NS
STREAM = 128
KSTREAMS = 80
NBUF = 3
NRING = 6
EPAD = NW * KSTREAMS * STREAM
NACC = 10016
RPS = 632
RPSL = NACC - (NS - 1) * RPS

RBLK = 2000
NBLK = N // RBLK
BN_EPS = 1e-5

_PREC = jax.lax.Precision.HIGHEST


def _sc_agg_body(y_hbm, srcdst_hbm, zeros_hbm, out_hbm,
                 ring, b0, b1, b2, acc_sh,
                 isems, gsems, ssems):
    cid = lax.axis_index("c")
    sid = lax.axis_index("s")
    wid = sid * NC + cid
    my_src = srcdst_hbm.at[0, wid]
    my_dst = srcdst_hbm.at[1, wid]

    bufs = [b0, b1, b2]

    @pl.when(sid < NS - 1)
    def _():
        pltpu.sync_copy(zeros_hbm, acc_sh.at[pl.ds(sid * RPS, RPS)])

    @pl.when(sid == NS - 1)
    def _():
        pltpu.sync_copy(zeros_hbm.at[pl.ds(0, RPSL)],
                        acc_sh.at[pl.ds((NS - 1) * RPS, RPSL)])

    plsc.subcore_barrier()

    def idxcopy(j, s):
        pltpu.async_copy(my_src.at[j], ring.at[s, 0], isems.at[s])
        pltpu.async_copy(my_dst.at[j], ring.at[s, 1], isems.at[s])

    def wait_idx(s):
        pltpu.make_async_copy(my_src.at[0], ring.at[s, 0], isems.at[s]).wait()
        pltpu.make_async_copy(my_dst.at[0], ring.at[s, 1], isems.at[s]).wait()

    def gather(s, t):
        pltpu.async_copy(y_hbm.at[ring.at[s, 0]], bufs[t], gsems.at[t])

    def wait_gather(t):
        pltpu.make_async_copy(y_hbm.at[ring.at[0, 0]], bufs[t],
                              gsems.at[t]).wait()

    def scat(s, t):
        pltpu.async_copy(bufs[t], acc_sh.at[ring.at[s, 1]], ssems.at[t],
                         add=True)

    def wait_scat(t):
        pltpu.make_async_copy(bufs[t], acc_sh.at[ring.at[0, 1]],
                              ssems.at[t]).wait()

    def step(j, k, sw=True, gat=True, idx=True):
        wait_gather(k % 3)
        if sw:
            wait_scat((k - 1) % 3)
        if gat:
            wait_idx((k + 2) % NRING)
            gather((k + 2) % NRING, (k + 2) % 3)
        scat(k % NRING, k % 3)
        if idx:
            idxcopy(j + 5, (k + 5) % NRING)

    for s in range(5):
        idxcopy(s, s)
    wait_idx(0)
    gather(0, 0)
    wait_idx(1)
    gather(1, 1)
    step(0, 0, sw=False)
    step(1, 1)

    @pl.loop(0, 12)
    def _(jj):
        j6 = 2 + jj * 6
        for k in range(6):
            step(j6 + k, 2 + k)

    for j in range(74, KSTREAMS):
        step(j, j, gat=(j + 2 < KSTREAMS), idx=(j + 5 < KSTREAMS))
    wait_scat((KSTREAMS - 1) % 3)

    plsc.subcore_barrier()

    @pl.when(sid < NS - 1)
    def _():
        pltpu.sync_copy(acc_sh.at[pl.ds(sid * RPS, RPS)],
                        out_hbm.at[cid].at[pl.ds(sid * RPS, RPS)])

    @pl.when(sid == NS - 1)
    def _():
        pltpu.sync_copy(acc_sh.at[pl.ds((NS - 1) * RPS, RPSL)],
                        out_hbm.at[cid].at[pl.ds((NS - 1) * RPS, RPSL)])


def _sc_agg(y, srcdst, zeros):
    mesh = plsc.VectorSubcoreMesh(core_axis_name="c", subcore_axis_name="s")
    kern = pl.kernel(
        _sc_agg_body,
        out_type=jax.ShapeDtypeStruct((NC, NACC, HP), jnp.float32),
        mesh=mesh,
        scratch_types=[
            pltpu.VMEM((NRING, 2, STREAM), jnp.int32),
            pltpu.VMEM((STREAM, HP), jnp.float32),
            pltpu.VMEM((STREAM, HP), jnp.float32),
            pltpu.VMEM((STREAM, HP), jnp.float32),
            pltpu.VMEM_SHARED((NACC, HP), jnp.float32),
            pltpu.SemaphoreType.DMA((NRING,)),
            pltpu.SemaphoreType.DMA((NBUF,)),
            pltpu.SemaphoreType.DMA((NBUF,)),
        ],
    )
    return kern(y, srcdst, zeros)


def _mlp_body(x_ref, agg_ref, w1_ref, b1_ref, w2_ref, b2_ref,
              r_ref, st_ref):
    i = pl.program_id(0)
    s = x_ref[...] + agg_ref[0] + agg_ref[1]
    h = jax.lax.dot_general(s, w1_ref[...], (((1,), (0,)), ((), ())),
                            precision=_PREC,
                            preferred_element_type=jnp.float32)
    h = jnp.maximum(h + b1_ref[...], 0.0)
    t = jax.lax.dot_general(h, w2_ref[...], (((1,), (0,)), ((), ())),
                            precision=_PREC,
                            preferred_element_type=jnp.float32)
    r = jnp.maximum(t + b2_ref[...], 0.0)
    col = jax.lax.broadcasted_iota(jnp.int32, (RBLK, H), 1)
    ones_lane = jnp.where(col == 0, 1.0, 0.0).astype(jnp.float32)
    r_ref[...] = jnp.concatenate([r, ones_lane], axis=1)

    @pl.when(i == 0)
    def _():
        st_ref[...] = jnp.zeros_like(st_ref)

    st_ref[0:1, :] += jnp.sum(r, axis=0, keepdims=True)
    st_ref[1:2, :] += jnp.sum(r * r, axis=0, keepdims=True)


def _mlp(x, agg, w1, b1, w2, b2):
    return pl.pallas_call(
        _mlp_body,
        grid=(NBLK,),
        in_specs=[
            pl.BlockSpec((RBLK, HP), lambda i: (i, 0)),
            pl.BlockSpec((NC, RBLK, HP), lambda i: (0, i, 0)),
            pl.BlockSpec((HP, H), lambda i: (0, 0)),
            pl.BlockSpec((1, H), lambda i: (0, 0)),
            pl.BlockSpec((H, H), lambda i: (0, 0)),
            pl.BlockSpec((1, H), lambda i: (0, 0)),
        ],
        out_specs=[
            pl.BlockSpec((RBLK, HP), lambda i: (i, 0)),
            pl.BlockSpec((8, H), lambda i: (0, 0)),
        ],
        out_shape=[
            jax.ShapeDtypeStruct((N, HP), jnp.float32),
            jax.ShapeDtypeStruct((8, H), jnp.float32),
        ],
    )(x, agg, w1, b1, w2, b2)


def _bn_coeffs(st_ref, gamma_ref, beta_ref):
    mu = st_ref[0:1, :] * (1.0 / N)
    var = st_ref[1:2, :] * (1.0 / N) - mu * mu
    c = gamma_ref[...] * jax.lax.rsqrt(var + BN_EPS)
    d = beta_ref[...] - mu * c
    return c, d


def _bn0_body(r_ref, st_ref, gamma_ref, beta_ref, w1_ref, z_ref, m_ref):
    c, d = _bn_coeffs(st_ref, gamma_ref, beta_ref)
    z_ref[...] = r_ref[:, :H] * c + d
    row = jax.lax.broadcasted_iota(jnp.int32, (H, H), 0)
    colj = jax.lax.broadcasted_iota(jnp.int32, (H, H), 1)
    eyec = jnp.where(row == colj, 1.0, 0.0).astype(jnp.float32) * c
    cw1 = jax.lax.dot_general(eyec, w1_ref[...], (((1,), (0,)), ((), ())),
                              precision=_PREC,
                              preferred_element_type=jnp.float32)
    dw1 = jax.lax.dot_general(d, w1_ref[...], (((1,), (0,)), ((), ())),
                              precision=_PREC,
                              preferred_element_type=jnp.float32)
    zpad = jnp.zeros((HP - H - 1, H), jnp.float32)
    m_ref[...] = jnp.concatenate([cw1, dw1, zpad], axis=0)


def _bn0(r, st, gamma, beta, w1):
    return pl.pallas_call(
        _bn0_body,
        grid=(NBLK,),
        in_specs=[
            pl.BlockSpec((RBLK, HP), lambda i: (i, 0)),
            pl.BlockSpec((8, H), lambda i: (0, 0)),
            pl.BlockSpec((1, H), lambda i: (0, 0)),
            pl.BlockSpec((1, H), lambda i: (0, 0)),
            pl.BlockSpec((H, H), lambda i: (0, 0)),
        ],
        out_specs=[
            pl.BlockSpec((RBLK, H), lambda i: (i, 0)),
            pl.BlockSpec((HP, H), lambda i: (0, 0)),
        ],
        out_shape=[
            jax.ShapeDtypeStruct((N, H), jnp.float32),
            jax.ShapeDtypeStruct((HP, H), jnp.float32),
        ],
    )(r, st, gamma, beta, w1)


def _final_body(r_ref, st_ref, gamma_ref, beta_ref, z0_ref, batch_ref,
                z_ref, g_ref):
    i = pl.program_id(0)
    c, d = _bn_coeffs(st_ref, gamma_ref, beta_ref)
    z1 = r_ref[:, :H] * c + d
    zc = jnp.concatenate([z0_ref[...], z1], axis=1)
    z_ref[...] = zc

    seg = batch_ref[...]
    cols = jax.lax.broadcasted_iota(jnp.int32, (RBLK, G), 1)
    p = jnp.where(seg == cols, 1.0, 0.0).astype(jnp.float32)
    gpart = jax.lax.dot_general(p, zc, (((0,), (0,)), ((), ())),
                                precision=_PREC,
                                preferred_element_type=jnp.float32)

    @pl.when(i == 0)
    def _():
        g_ref[...] = jnp.zeros_like(g_ref)

    g_ref[...] += gpart


def _final(r, st, gamma, beta, z0, batch2d):
    return pl.pallas_call(
        _final_body,
        grid=(NBLK,),
        in_specs=[
            pl.BlockSpec((RBLK, HP), lambda i: (i, 0)),
            pl.BlockSpec((8, H), lambda i: (0, 0)),
            pl.BlockSpec((1, H), lambda i: (0, 0)),
            pl.BlockSpec((1, H), lambda i: (0, 0)),
            pl.BlockSpec((RBLK, H), lambda i: (i, 0)),
            pl.BlockSpec((RBLK, 1), lambda i: (i, 0)),
        ],
        out_specs=[
            pl.BlockSpec((RBLK, 2 * H), lambda i: (i, 0)),
            pl.BlockSpec((G, 2 * H), lambda i: (0, 0)),
        ],
        out_shape=[
            jax.ShapeDtypeStruct((N, 2 * H), jnp.float32),
            jax.ShapeDtypeStruct((G, 2 * H), jnp.float32),
        ],
    )(r, st, gamma, beta, z0, batch2d)


def kernel(x, edge_index, batch, W1_0, b1_0, W2_0, b2_0, gamma_0, beta_0,
           W1_1, b1_1, W2_1, b2_1, gamma_1, beta_1):
    pad = EPAD - E
    pad_ar = jnp.arange(pad, dtype=jnp.int32)
    src_pad = (pad_ar * 37) % N
    dst_pad = N + (pad_ar % (NACC - N))
    srcdst = jnp.concatenate(
        [edge_index, jnp.stack([src_pad, dst_pad])], axis=1).reshape(
        2, NW, KSTREAMS, STREAM)
    zeros = jnp.zeros((RPS, HP), jnp.float32)
    batch2d = batch.reshape(N, 1)
    b1_0r, b2_0r = b1_0.reshape(1, H), b2_0.reshape(1, H)
    b1_1r, b2_1r = b1_1.reshape(1, H), b2_1.reshape(1, H)
    g0r, be0r = gamma_0.reshape(1, H), beta_0.reshape(1, H)
    g1r, be1r = gamma_1.reshape(1, H), beta_1.reshape(1, H)

    agg0 = _sc_agg(x, srcdst, zeros)
    r0p, st0 = _mlp(x, agg0, W1_0, b1_0r, W2_0, b2_0r)

    agg1 = _sc_agg(r0p, srcdst, zeros)
    z0, m0 = _bn0(r0p, st0, g0r, be0r, W1_1)
    r1p, st1 = _mlp(r0p, agg1, m0, b1_1r, W2_1, b2_1r)

    z_out, g_out = _final(r1p, st1, g1r, be1r, z0, batch2d)
    return (z_out, g_out)


# speedup vs baseline: 1.0541x; 1.0161x over previous
import jax
import jax.numpy as jnp
from jax import lax
from jax.experimental import pallas as pl
from jax.experimental.pallas import tpu as pltpu
from jax.experimental.pallas import tpu_sc as plsc

N = 10000
D = 128
H = 64
HP = 128
G = 128
E = 320000

NC = 2
NS = 16
NW = NC * NS
STREAM = 128
KSTREAMS = 80
NBUF = 3
NRING = 6
EPAD = NW * KSTREAMS * STREAM
NACC = 10016
RPS = 632
RPSL = NACC - (NS - 1) * RPS

RBLK = 2000
NBLK = N // RBLK
BN_EPS = 1e-5

_PREC = jax.lax.Precision.HIGHEST


def _sc_agg_body(y_hbm, srcdst_hbm, zeros_hbm, out_hbm,
                 ring, b0, b1, b2, acc_sh,
                 isems, gsems, ssems):
    cid = lax.axis_index("c")
    sid = lax.axis_index("s")
    wid = sid * NC + cid
    my_src = srcdst_hbm.at[0, wid]
    my_dst = srcdst_hbm.at[1, wid]

    bufs = [b0, b1, b2]

    def idxcopy(j, s):
        pltpu.async_copy(my_src.at[j], ring.at[s, 0], isems.at[s])
        pltpu.async_copy(my_dst.at[j], ring.at[s, 1], isems.at[s])

    def wait_idx(s):
        pltpu.make_async_copy(my_src.at[0], ring.at[s, 0], isems.at[s]).wait()
        pltpu.make_async_copy(my_dst.at[0], ring.at[s, 1], isems.at[s]).wait()

    def gather(s, t):
        pltpu.async_copy(y_hbm.at[ring.at[s, 0]], bufs[t], gsems.at[t])

    def wait_gather(t):
        pltpu.make_async_copy(y_hbm.at[ring.at[0, 0]], bufs[t],
                              gsems.at[t]).wait()

    def scat(s, t):
        pltpu.async_copy(bufs[t], acc_sh.at[ring.at[s, 1]], ssems.at[t],
                         add=True)

    def wait_scat(t):
        pltpu.make_async_copy(bufs[t], acc_sh.at[ring.at[0, 1]],
                              ssems.at[t]).wait()

    def step(j, k, sw=True, gat=True, idx=True):
        wait_gather(k % 3)
        if sw:
            wait_scat((k - 1) % 3)
        if gat:
            wait_idx((k + 2) % NRING)
            gather((k + 2) % NRING, (k + 2) % 3)
        scat(k % NRING, k % 3)
        if idx:
            idxcopy(j + 5, (k + 5) % NRING)

    for s in range(5):
        idxcopy(s, s)
    wait_idx(0)
    gather(0, 0)
    wait_idx(1)
    gather(1, 1)

    @pl.when(sid < NS - 1)
    def _():
        pltpu.sync_copy(zeros_hbm, acc_sh.at[pl.ds(sid * RPS, RPS)])

    @pl.when(sid == NS - 1)
    def _():
        pltpu.sync_copy(zeros_hbm.at[pl.ds(0, RPSL)],
                        acc_sh.at[pl.ds((NS - 1) * RPS, RPSL)])

    plsc.subcore_barrier()

    step(0, 0, sw=False)
    step(1, 1)

    @pl.loop(0, 12)
    def _(jj):
        j6 = 2 + jj * 6
        for k in range(6):
            step(j6 + k, 2 + k)

    for j in range(74, KSTREAMS):
        step(j, j, gat=(j + 2 < KSTREAMS), idx=(j + 5 < KSTREAMS))
    wait_scat((KSTREAMS - 1) % 3)

    plsc.subcore_barrier()

    @pl.when(sid < NS - 1)
    def _():
        pltpu.sync_copy(acc_sh.at[pl.ds(sid * RPS, RPS)],
                        out_hbm.at[cid].at[pl.ds(sid * RPS, RPS)])

    @pl.when(sid == NS - 1)
    def _():
        pltpu.sync_copy(acc_sh.at[pl.ds((NS - 1) * RPS, RPSL)],
                        out_hbm.at[cid].at[pl.ds((NS - 1) * RPS, RPSL)])


def _sc_agg(y, srcdst, zeros):
    mesh = plsc.VectorSubcoreMesh(core_axis_name="c", subcore_axis_name="s")
    kern = pl.kernel(
        _sc_agg_body,
        out_type=jax.ShapeDtypeStruct((NC, NACC, HP), jnp.float32),
        mesh=mesh,
        scratch_types=[
            pltpu.VMEM((NRING, 2, STREAM), jnp.int32),
            pltpu.VMEM((STREAM, HP), jnp.float32),
            pltpu.VMEM((STREAM, HP), jnp.float32),
            pltpu.VMEM((STREAM, HP), jnp.float32),
            pltpu.VMEM_SHARED((NACC, HP), jnp.float32),
            pltpu.SemaphoreType.DMA((NRING,)),
            pltpu.SemaphoreType.DMA((NBUF,)),
            pltpu.SemaphoreType.DMA((NBUF,)),
        ],
    )
    return kern(y, srcdst, zeros)


def _mlp_body(x_ref, agg_ref, w1_ref, b1_ref, w2_ref, b2_ref,
              r_ref, st_ref):
    i = pl.program_id(0)
    s = x_ref[...] + agg_ref[0] + agg_ref[1]
    h = jax.lax.dot_general(s, w1_ref[...], (((1,), (0,)), ((), ())),
                            precision=_PREC,
                            preferred_element_type=jnp.float32)
    h = jnp.maximum(h + b1_ref[...], 0.0)
    t = jax.lax.dot_general(h, w2_ref[...], (((1,), (0,)), ((), ())),
                            precision=_PREC,
                            preferred_element_type=jnp.float32)
    r = jnp.maximum(t + b2_ref[...], 0.0)
    col = jax.lax.broadcasted_iota(jnp.int32, (RBLK, H), 1)
    ones_lane = jnp.where(col == 0, 1.0, 0.0).astype(jnp.float32)
    r_ref[...] = jnp.concatenate([r, ones_lane], axis=1)

    @pl.when(i == 0)
    def _():
        st_ref[...] = jnp.zeros_like(st_ref)

    st_ref[0:1, :] += jnp.sum(r, axis=0, keepdims=True)
    st_ref[1:2, :] += jnp.sum(r * r, axis=0, keepdims=True)


def _mlp(x, agg, w1, b1, w2, b2):
    return pl.pallas_call(
        _mlp_body,
        grid=(NBLK,),
        in_specs=[
            pl.BlockSpec((RBLK, HP), lambda i: (i, 0)),
            pl.BlockSpec((NC, RBLK, HP), lambda i: (0, i, 0)),
            pl.BlockSpec((HP, H), lambda i: (0, 0)),
            pl.BlockSpec((1, H), lambda i: (0, 0)),
            pl.BlockSpec((H, H), lambda i: (0, 0)),
            pl.BlockSpec((1, H), lambda i: (0, 0)),
        ],
        out_specs=[
            pl.BlockSpec((RBLK, HP), lambda i: (i, 0)),
            pl.BlockSpec((8, H), lambda i: (0, 0)),
        ],
        out_shape=[
            jax.ShapeDtypeStruct((N, HP), jnp.float32),
            jax.ShapeDtypeStruct((8, H), jnp.float32),
        ],
    )(x, agg, w1, b1, w2, b2)


def _bn_coeffs(st_ref, gamma_ref, beta_ref):
    mu = st_ref[0:1, :] * (1.0 / N)
    var = st_ref[1:2, :] * (1.0 / N) - mu * mu
    c = gamma_ref[...] * jax.lax.rsqrt(var + BN_EPS)
    d = beta_ref[...] - mu * c
    return c, d


def _bn0_body(r_ref, st_ref, gamma_ref, beta_ref, w1_ref, z_ref, m_ref):
    c, d = _bn_coeffs(st_ref, gamma_ref, beta_ref)
    z_ref[...] = r_ref[:, :H] * c + d
    row = jax.lax.broadcasted_iota(jnp.int32, (H, H), 0)
    colj = jax.lax.broadcasted_iota(jnp.int32, (H, H), 1)
    eyec = jnp.where(row == colj, 1.0, 0.0).astype(jnp.float32) * c
    cw1 = jax.lax.dot_general(eyec, w1_ref[...], (((1,), (0,)), ((), ())),
                              precision=_PREC,
                              preferred_element_type=jnp.float32)
    dw1 = jax.lax.dot_general(d, w1_ref[...], (((1,), (0,)), ((), ())),
                              precision=_PREC,
                              preferred_element_type=jnp.float32)
    zpad = jnp.zeros((HP - H - 1, H), jnp.float32)
    m_ref[...] = jnp.concatenate([cw1, dw1, zpad], axis=0)


def _bn0(r, st, gamma, beta, w1):
    return pl.pallas_call(
        _bn0_body,
        grid=(NBLK,),
        in_specs=[
            pl.BlockSpec((RBLK, HP), lambda i: (i, 0)),
            pl.BlockSpec((8, H), lambda i: (0, 0)),
            pl.BlockSpec((1, H), lambda i: (0, 0)),
            pl.BlockSpec((1, H), lambda i: (0, 0)),
            pl.BlockSpec((H, H), lambda i: (0, 0)),
        ],
        out_specs=[
            pl.BlockSpec((RBLK, H), lambda i: (i, 0)),
            pl.BlockSpec((HP, H), lambda i: (0, 0)),
        ],
        out_shape=[
            jax.ShapeDtypeStruct((N, H), jnp.float32),
            jax.ShapeDtypeStruct((HP, H), jnp.float32),
        ],
    )(r, st, gamma, beta, w1)


def _final_body(r_ref, st_ref, gamma_ref, beta_ref, z0_ref, batch_ref,
                z_ref, g_ref):
    i = pl.program_id(0)
    c, d = _bn_coeffs(st_ref, gamma_ref, beta_ref)
    z1 = r_ref[:, :H] * c + d
    zc = jnp.concatenate([z0_ref[...], z1], axis=1)
    z_ref[...] = zc

    seg = batch_ref[...]
    cols = jax.lax.broadcasted_iota(jnp.int32, (RBLK, G), 1)
    p = jnp.where(seg == cols, 1.0, 0.0).astype(jnp.float32)
    gpart = jax.lax.dot_general(p, zc, (((0,), (0,)), ((), ())),
                                precision=_PREC,
                                preferred_element_type=jnp.float32)

    @pl.when(i == 0)
    def _():
        g_ref[...] = jnp.zeros_like(g_ref)

    g_ref[...] += gpart


def _final(r, st, gamma, beta, z0, batch2d):
    return pl.pallas_call(
        _final_body,
        grid=(NBLK,),
        in_specs=[
            pl.BlockSpec((RBLK, HP), lambda i: (i, 0)),
            pl.BlockSpec((8, H), lambda i: (0, 0)),
            pl.BlockSpec((1, H), lambda i: (0, 0)),
            pl.BlockSpec((1, H), lambda i: (0, 0)),
            pl.BlockSpec((RBLK, H), lambda i: (i, 0)),
            pl.BlockSpec((RBLK, 1), lambda i: (i, 0)),
        ],
        out_specs=[
            pl.BlockSpec((RBLK, 2 * H), lambda i: (i, 0)),
            pl.BlockSpec((G, 2 * H), lambda i: (0, 0)),
        ],
        out_shape=[
            jax.ShapeDtypeStruct((N, 2 * H), jnp.float32),
            jax.ShapeDtypeStruct((G, 2 * H), jnp.float32),
        ],
    )(r, st, gamma, beta, z0, batch2d)


def kernel(x, edge_index, batch, W1_0, b1_0, W2_0, b2_0, gamma_0, beta_0,
           W1_1, b1_1, W2_1, b2_1, gamma_1, beta_1):
    pad = EPAD - E
    pad_ar = jnp.arange(pad, dtype=jnp.int32)
    src_pad = (pad_ar * 37) % N
    dst_pad = N + (pad_ar % (NACC - N))
    srcdst = jnp.concatenate(
        [edge_index, jnp.stack([src_pad, dst_pad])], axis=1).reshape(
        2, NW, KSTREAMS, STREAM)
    zeros = jnp.zeros((RPS, HP), jnp.float32)
    batch2d = batch.reshape(N, 1)
    b1_0r, b2_0r = b1_0.reshape(1, H), b2_0.reshape(1, H)
    b1_1r, b2_1r = b1_1.reshape(1, H), b2_1.reshape(1, H)
    g0r, be0r = gamma_0.reshape(1, H), beta_0.reshape(1, H)
    g1r, be1r = gamma_1.reshape(1, H), beta_1.reshape(1, H)

    agg0 = _sc_agg(x, srcdst, zeros)
    r0p, st0 = _mlp(x, agg0, W1_0, b1_0r, W2_0, b2_0r)

    agg1 = _sc_agg(r0p, srcdst, zeros)
    z0, m0 = _bn0(r0p, st0, g0r, be0r, W1_1)
    r1p, st1 = _mlp(r0p, agg1, m0, b1_1r, W2_1, b2_1r)

    z_out, g_out = _final(r1p, st1, g1r, be1r, z0, batch2d)
    return (z_out, g_out)
